# baseline (speedup 1.0000x reference)
import functools

import jax
import jax.numpy as jnp
from jax import lax
from jax.experimental import pallas as pl
from jax.experimental.pallas import tpu as pltpu
from jax.experimental.pallas import tpu_sc as plsc

_VOCAB = 100000
_DIM = 128
_BATCH = 16384
_NC = 2
_NS = 16
_NW = _NC * _NS
_BW = _BATCH // _NW
_CHUNK = 128
_NCHUNK = _BW // _CHUNK
_NBUF = 3
_L = 16
_GPC = _CHUNK // _L
_NGROUP = _BW // _L


def _body(center_hbm, context_hbm, wc_hbm, wx_hbm, out_hbm,
          cidx_v, xidx_v, crows_v, xrows_v, out_v, isem, csem, xsem):
    wid = lax.axis_index("s") * _NC + lax.axis_index("c")
    base = wid * _BW

    idx_copies = []
    for src, dst in ((center_hbm, cidx_v), (context_hbm, xidx_v)):
        cp = pltpu.make_async_copy(src.at[pl.ds(base, _BW)], dst, isem)
        cp.start()
        idx_copies.append(cp)
    for cp in idx_copies:
        cp.wait()

    def gather_pair(c, buf):
        return (
            pltpu.make_async_copy(
                wc_hbm.at[cidx_v.at[pl.ds(c * _CHUNK, _CHUNK)]],
                crows_v.at[buf], csem.at[buf]),
            pltpu.make_async_copy(
                wx_hbm.at[xidx_v.at[pl.ds(c * _CHUNK, _CHUNK)]],
                xrows_v.at[buf], xsem.at[buf]),
        )

    for c in range(_NBUF - 1):
        for cp in gather_pair(c, c):
            cp.start()

    lanes = lax.iota(jnp.int32, _L)
    perms = [lanes ^ d for d in (8, 4, 2, 1)]
    merge_masks = [(lanes & d) == 0 for d in (8, 4, 2, 1)]

    def shuf(v, p):
        return v.at[p].get(mode="promise_in_bounds")

    def group(g, _):
        c = g // _GPC
        buf = c % _NBUF

        @pl.when(g % _GPC == 0)
        def _boundary():
            nc = c + _NBUF - 1

            @pl.when(nc < _NCHUNK)
            def _start_next():
                for cp in gather_pair(nc, nc % _NBUF):
                    cp.start()

            for cp in gather_pair(c, buf):
                cp.wait()

        row0 = (g % _GPC) * _L
        accs = []
        for i in range(_L):
            row = row0 + i
            acc = (crows_v[buf, row, pl.ds(0, _L)] *
                   xrows_v[buf, row, pl.ds(0, _L)])
            for j in range(1, _DIM // _L):
                acc = acc + (crows_v[buf, row, pl.ds(j * _L, _L)] *
                             xrows_v[buf, row, pl.ds(j * _L, _L)])
            accs.append(acc)
        for p, m in zip(perms, merge_masks):
            nxt = []
            half = len(accs) // 2
            for i in range(half):
                a = accs[i]
                b = accs[i + half]
                nxt.append(jnp.where(m, a + shuf(a, p), b + shuf(b, p)))
            accs = nxt
        out_v[pl.ds(g * _L, _L)] = accs[0]
        return 0

    lax.fori_loop(0, _NGROUP, group, 0, unroll=4)

    pltpu.sync_copy(out_v, out_hbm.at[pl.ds(base, _BW)])


@jax.jit
def _run(center, context, W_center, W_context):
    mesh = plsc.VectorSubcoreMesh(core_axis_name="c", subcore_axis_name="s")
    k = functools.partial(
        pl.kernel,
        mesh=mesh,
        out_type=jax.ShapeDtypeStruct((_BATCH,), jnp.float32),
        scratch_types=[
            pltpu.VMEM((_BW,), jnp.int32),
            pltpu.VMEM((_BW,), jnp.int32),
            pltpu.VMEM((_NBUF, _CHUNK, _DIM), jnp.float32),
            pltpu.VMEM((_NBUF, _CHUNK, _DIM), jnp.float32),
            pltpu.VMEM((_BW,), jnp.float32),
            pltpu.SemaphoreType.DMA,
            pltpu.SemaphoreType.DMA((_NBUF,)),
            pltpu.SemaphoreType.DMA((_NBUF,)),
        ],
    )(_body)
    return k(center, context, W_center, W_context)


def kernel(center, context, W_center, W_context):
    return _run(center, context, W_center, W_context)


# speedup vs baseline: 1.3854x; 1.3854x over previous
import functools

import jax
import jax.numpy as jnp
from jax import lax
from jax.experimental import pallas as pl
from jax.experimental.pallas import tpu as pltpu
from jax.experimental.pallas import tpu_sc as plsc

_VOCAB = 100000
_DIM = 128
_BATCH = 16384
_NC = 2
_NS = 16
_NW = _NC * _NS
_BW = _BATCH // _NW
_CHUNK = 64
_NCHUNK = _BW // _CHUNK
_NBUF = 6
_L = 16
_GPC = _CHUNK // _L
_NGROUP = _BW // _L


def _body(center_hbm, context_hbm, wc_hbm, wx_hbm, out_hbm,
          cidx_v, xidx_v, crows_v, xrows_v, out_v, isem, csem, xsem):
    wid = lax.axis_index("s") * _NC + lax.axis_index("c")
    base = wid * _BW

    idx_copies = []
    for c in range(_NCHUNK):
        for src, dst in ((center_hbm, cidx_v), (context_hbm, xidx_v)):
            cp = pltpu.make_async_copy(
                src.at[pl.ds(base + c * _CHUNK, _CHUNK)], dst.at[c], isem)
            cp.start()
            idx_copies.append(cp)
    for cp in idx_copies:
        cp.wait()

    def gather_pair(c, buf):
        return (
            pltpu.make_async_copy(wc_hbm.at[cidx_v.at[c]],
                                  crows_v.at[buf], csem.at[buf]),
            pltpu.make_async_copy(wx_hbm.at[xidx_v.at[c]],
                                  xrows_v.at[buf], xsem.at[buf]),
        )

    for c in range(_NBUF - 1):
        for cp in gather_pair(c, c):
            cp.start()

    lanes = lax.iota(jnp.int32, _L)
    perms = [lanes ^ d for d in (8, 4, 2, 1)]
    merge_masks = [(lanes & d) == 0 for d in (8, 4, 2, 1)]

    def shuf(v, p):
        return v.at[p].get(mode="promise_in_bounds")

    def group(g, _):
        c = g // _GPC
        buf = c % _NBUF

        @pl.when(g % _GPC == 0)
        def _boundary():
            nc = c + _NBUF - 1

            @pl.when(nc < _NCHUNK)
            def _start_next():
                for cp in gather_pair(nc, nc % _NBUF):
                    cp.start()

            for cp in gather_pair(c, buf):
                cp.wait()

        row0 = (g % _GPC) * _L
        accs = []
        for i in range(_L):
            row = row0 + i
            acc = (crows_v[buf, row, pl.ds(0, _L)] *
                   xrows_v[buf, row, pl.ds(0, _L)])
            for j in range(1, _DIM // _L):
                acc = acc + (crows_v[buf, row, pl.ds(j * _L, _L)] *
                             xrows_v[buf, row, pl.ds(j * _L, _L)])
            accs.append(acc)
        for p, m in zip(perms, merge_masks):
            nxt = []
            half = len(accs) // 2
            for i in range(half):
                a = accs[i]
                b = accs[i + half]
                nxt.append(jnp.where(m, a + shuf(a, p), b + shuf(b, p)))
            accs = nxt
        out_v[pl.ds(g * _L, _L)] = accs[0]
        return 0

    lax.fori_loop(0, _NGROUP, group, 0, unroll=2)

    pltpu.sync_copy(out_v, out_hbm.at[pl.ds(base, _BW)])


@jax.jit
def _run(center, context, W_center, W_context):
    mesh = plsc.VectorSubcoreMesh(core_axis_name="c", subcore_axis_name="s")
    k = functools.partial(
        pl.kernel,
        mesh=mesh,
        out_type=jax.ShapeDtypeStruct((_BATCH,), jnp.float32),
        scratch_types=[
            pltpu.VMEM((_NCHUNK, _CHUNK), jnp.int32),
            pltpu.VMEM((_NCHUNK, _CHUNK), jnp.int32),
            pltpu.VMEM((_NBUF, _CHUNK, _DIM), jnp.float32),
            pltpu.VMEM((_NBUF, _CHUNK, _DIM), jnp.float32),
            pltpu.VMEM((_BW,), jnp.float32),
            pltpu.SemaphoreType.DMA,
            pltpu.SemaphoreType.DMA((_NBUF,)),
            pltpu.SemaphoreType.DMA((_NBUF,)),
        ],
    )(_body)
    return k(center, context, W_center, W_context)


def kernel(center, context, W_center, W_context):
    return _run(center, context, W_center, W_context)


# speedup vs baseline: 1.3887x; 1.0023x over previous
import functools

import jax
import jax.numpy as jnp
from jax import lax
from jax.experimental import pallas as pl
from jax.experimental.pallas import tpu as pltpu
from jax.experimental.pallas import tpu_sc as plsc

_VOCAB = 100000
_DIM = 128
_BATCH = 16384
_NC = 2
_NS = 16
_NW = _NC * _NS
_BW = _BATCH // _NW
_CHUNK = 64
_NCHUNK = _BW // _CHUNK
_NBUF = 7
_L = 16
_GPC = _CHUNK // _L
_NGROUP = _BW // _L


def _body(center_hbm, context_hbm, wc_hbm, wx_hbm, out_hbm,
          cidx_v, xidx_v, crows_v, xrows_v, out_v, isem, csem, xsem):
    wid = lax.axis_index("s") * _NC + lax.axis_index("c")
    base = wid * _BW

    idx_copies = []
    for c in range(_NCHUNK):
        for src, dst in ((center_hbm, cidx_v), (context_hbm, xidx_v)):
            cp = pltpu.make_async_copy(
                src.at[pl.ds(base + c * _CHUNK, _CHUNK)], dst.at[c], isem)
            cp.start()
            idx_copies.append(cp)
    for cp in idx_copies:
        cp.wait()

    def gather_pair(c, buf):
        return (
            pltpu.make_async_copy(wc_hbm.at[cidx_v.at[c]],
                                  crows_v.at[buf], csem.at[buf]),
            pltpu.make_async_copy(wx_hbm.at[xidx_v.at[c]],
                                  xrows_v.at[buf], xsem.at[buf]),
        )

    for c in range(_NBUF - 1):
        for cp in gather_pair(c, c):
            cp.start()

    lanes = lax.iota(jnp.int32, _L)
    perms = [lanes ^ d for d in (8, 4, 2, 1)]
    merge_masks = [(lanes & d) == 0 for d in (8, 4, 2, 1)]

    def shuf(v, p):
        return v.at[p].get(mode="promise_in_bounds")

    def group(g, _):
        c = g // _GPC
        buf = c % _NBUF

        @pl.when(g % _GPC == 0)
        def _boundary():
            nc = c + _NBUF - 1

            @pl.when(nc < _NCHUNK)
            def _start_next():
                for cp in gather_pair(nc, nc % _NBUF):
                    cp.start()

            for cp in gather_pair(c, buf):
                cp.wait()

        row0 = (g % _GPC) * _L
        accs = []
        for i in range(_L):
            row = row0 + i
            acc = (crows_v[buf, row, pl.ds(0, _L)] *
                   xrows_v[buf, row, pl.ds(0, _L)])
            for j in range(1, _DIM // _L):
                acc = acc + (crows_v[buf, row, pl.ds(j * _L, _L)] *
                             xrows_v[buf, row, pl.ds(j * _L, _L)])
            accs.append(acc)
        for p, m in zip(perms, merge_masks):
            nxt = []
            half = len(accs) // 2
            for i in range(half):
                a = accs[i]
                b = accs[i + half]
                nxt.append(jnp.where(m, a + shuf(a, p), b + shuf(b, p)))
            accs = nxt
        out_v[pl.ds(g * _L, _L)] = accs[0]
        return 0

    lax.fori_loop(0, _NGROUP, group, 0, unroll=2)

    pltpu.sync_copy(out_v, out_hbm.at[pl.ds(base, _BW)])


@jax.jit
def _run(center, context, W_center, W_context):
    mesh = plsc.VectorSubcoreMesh(core_axis_name="c", subcore_axis_name="s")
    k = functools.partial(
        pl.kernel,
        mesh=mesh,
        out_type=jax.ShapeDtypeStruct((_BATCH,), jnp.float32),
        scratch_types=[
            pltpu.VMEM((_NCHUNK, _CHUNK), jnp.int32),
            pltpu.VMEM((_NCHUNK, _CHUNK), jnp.int32),
            pltpu.VMEM((_NBUF, _CHUNK, _DIM), jnp.float32),
            pltpu.VMEM((_NBUF, _CHUNK, _DIM), jnp.float32),
            pltpu.VMEM((_BW,), jnp.float32),
            pltpu.SemaphoreType.DMA,
            pltpu.SemaphoreType.DMA((_NBUF,)),
            pltpu.SemaphoreType.DMA((_NBUF,)),
        ],
    )(_body)
    return k(center, context, W_center, W_context)


def kernel(center, context, W_center, W_context):
    return _run(center, context, W_center, W_context)


# speedup vs baseline: 1.4151x; 1.0190x over previous
import functools

import jax
import jax.numpy as jnp
from jax import lax
from jax.experimental import pallas as pl
from jax.experimental.pallas import tpu as pltpu
from jax.experimental.pallas import tpu_sc as plsc

_VOCAB = 100000
_DIM = 128
_BATCH = 16384
_NC = 2
_NS = 16
_NW = _NC * _NS
_BW = _BATCH // _NW
_CHUNK = 32
_NCHUNK = _BW // _CHUNK
_NBUF = 12
_L = 16
_GPC = _CHUNK // _L
_NGROUP = _BW // _L


def _body(center_hbm, context_hbm, wc_hbm, wx_hbm, out_hbm,
          cidx_v, xidx_v, crows_v, xrows_v, out_v, isem, csem, xsem):
    wid = lax.axis_index("s") * _NC + lax.axis_index("c")
    base = wid * _BW

    idx_copies = []
    for c in range(_NCHUNK):
        for src, dst in ((center_hbm, cidx_v), (context_hbm, xidx_v)):
            cp = pltpu.make_async_copy(
                src.at[pl.ds(base + c * _CHUNK, _CHUNK)], dst.at[c], isem)
            cp.start()
            idx_copies.append(cp)
    for cp in idx_copies:
        cp.wait()

    def gather_pair(c, buf):
        return (
            pltpu.make_async_copy(wc_hbm.at[cidx_v.at[c]],
                                  crows_v.at[buf], csem.at[buf]),
            pltpu.make_async_copy(wx_hbm.at[xidx_v.at[c]],
                                  xrows_v.at[buf], xsem.at[buf]),
        )

    for c in range(_NBUF - 1):
        for cp in gather_pair(c, c):
            cp.start()

    lanes = lax.iota(jnp.int32, _L)
    perms = [lanes ^ d for d in (8, 4, 2, 1)]
    merge_masks = [(lanes & d) == 0 for d in (8, 4, 2, 1)]

    def shuf(v, p):
        return v.at[p].get(mode="promise_in_bounds")

    def group(g, _):
        c = g // _GPC
        buf = c % _NBUF

        @pl.when(g % _GPC == 0)
        def _boundary():
            nc = c + _NBUF - 1

            @pl.when(nc < _NCHUNK)
            def _start_next():
                for cp in gather_pair(nc, nc % _NBUF):
                    cp.start()

            for cp in gather_pair(c, buf):
                cp.wait()

        row0 = (g % _GPC) * _L
        accs = []
        for i in range(_L):
            row = row0 + i
            acc = (crows_v[buf, row, pl.ds(0, _L)] *
                   xrows_v[buf, row, pl.ds(0, _L)])
            for j in range(1, _DIM // _L):
                acc = acc + (crows_v[buf, row, pl.ds(j * _L, _L)] *
                             xrows_v[buf, row, pl.ds(j * _L, _L)])
            accs.append(acc)
        for p, m in zip(perms, merge_masks):
            nxt = []
            half = len(accs) // 2
            for i in range(half):
                a = accs[i]
                b = accs[i + half]
                nxt.append(jnp.where(m, a + shuf(a, p), b + shuf(b, p)))
            accs = nxt
        out_v[pl.ds(g * _L, _L)] = accs[0]
        return 0

    lax.fori_loop(0, _NGROUP, group, 0, unroll=2)

    pltpu.sync_copy(out_v, out_hbm.at[pl.ds(base, _BW)])


@jax.jit
def _run(center, context, W_center, W_context):
    mesh = plsc.VectorSubcoreMesh(core_axis_name="c", subcore_axis_name="s")
    k = functools.partial(
        pl.kernel,
        mesh=mesh,
        out_type=jax.ShapeDtypeStruct((_BATCH,), jnp.float32),
        scratch_types=[
            pltpu.VMEM((_NCHUNK, _CHUNK), jnp.int32),
            pltpu.VMEM((_NCHUNK, _CHUNK), jnp.int32),
            pltpu.VMEM((_NBUF, _CHUNK, _DIM), jnp.float32),
            pltpu.VMEM((_NBUF, _CHUNK, _DIM), jnp.float32),
            pltpu.VMEM((_BW,), jnp.float32),
            pltpu.SemaphoreType.DMA,
            pltpu.SemaphoreType.DMA((_NBUF,)),
            pltpu.SemaphoreType.DMA((_NBUF,)),
        ],
    )(_body)
    return k(center, context, W_center, W_context)


def kernel(center, context, W_center, W_context):
    return _run(center, context, W_center, W_context)
